# Initial kernel scaffold; baseline (speedup 1.0000x reference)
import jax
import jax.numpy as jnp
from jax.experimental import pallas as pl


def kernel(x, edge_index, W1, b1, W2, b2):
    raise NotImplementedError("write your pallas kernel here")


# speedup vs baseline: 15.9469x; 15.9469x over previous
import functools

import jax
import jax.numpy as jnp
from jax import lax
from jax.experimental import pallas as pl
from jax.experimental.pallas import tpu as pltpu
from jax.experimental.pallas import tpu_sc as plsc

N_NODES = 10000
N_EDGES = 320000
NPAD = 10240
F = 16
NUM_SC = 2
TILES = 16
WORKERS = NUM_SC * TILES
EPT = N_EDGES // WORKERS
CH = 80
RPT = NPAD // TILES


def _deg_body(dst_hbm, zeros1_hbm, deg_out, acc_sh, idx_v, ones_v):
    c = lax.axis_index("c")
    s = lax.axis_index("s")
    wid = c * TILES + s
    for i in range(CH // 16):
        ones_v[pl.ds(i * 16, 16)] = jnp.ones((16,), jnp.float32)
    pltpu.sync_copy(zeros1_hbm.at[pl.ds(s * RPT, RPT)],
                    acc_sh.at[pl.ds(s * RPT, RPT)])
    plsc.subcore_barrier()

    def chunk(i, carry):
        base = wid * EPT + i * CH
        pltpu.sync_copy(dst_hbm.at[pl.ds(base, CH)], idx_v)
        pltpu.sync_copy(ones_v, acc_sh.at[idx_v], add=True)
        return carry

    lax.fori_loop(0, EPT // CH, chunk, 0)
    plsc.subcore_barrier()
    pltpu.sync_copy(acc_sh.at[pl.ds(s * RPT, RPT)],
                    deg_out.at[c, pl.ds(s * RPT, RPT)])


def _agg_body(hp_hbm, src_hbm, dst_hbm, zeros16_hbm, agg_out,
              acc_sh, sidx_v, didx_v, rows_v, sem):
    c = lax.axis_index("c")
    s = lax.axis_index("s")
    wid = c * TILES + s
    pltpu.sync_copy(zeros16_hbm.at[pl.ds(s * RPT, RPT)],
                    acc_sh.at[pl.ds(s * RPT, RPT)])
    plsc.subcore_barrier()

    def chunk(i, carry):
        base = wid * EPT + i * CH
        pltpu.sync_copy(src_hbm.at[pl.ds(base, CH)], sidx_v)
        pltpu.sync_copy(dst_hbm.at[pl.ds(base, CH)], didx_v)
        pltpu.async_copy(hp_hbm.at[sidx_v], rows_v, sem).wait()
        pltpu.sync_copy(rows_v, acc_sh.at[didx_v], add=True)
        return carry

    lax.fori_loop(0, EPT // CH, chunk, 0)
    plsc.subcore_barrier()
    pltpu.sync_copy(acc_sh.at[pl.ds(s * RPT, RPT)],
                    agg_out.at[c, pl.ds(s * RPT, RPT)])


@functools.cache
def _sc_calls():
    mesh = plsc.VectorSubcoreMesh(core_axis_name="c", subcore_axis_name="s",
                                  num_cores=NUM_SC, num_subcores=TILES)
    deg_call = pl.kernel(
        _deg_body,
        out_type=jax.ShapeDtypeStruct((NUM_SC, NPAD), jnp.float32),
        mesh=mesh,
        scratch_types=[
            pltpu.VMEM_SHARED((NPAD,), jnp.float32),
            pltpu.VMEM((CH,), jnp.int32),
            pltpu.VMEM((CH,), jnp.float32),
        ],
    )
    agg_call = pl.kernel(
        _agg_body,
        out_type=jax.ShapeDtypeStruct((NUM_SC, NPAD, F), jnp.float32),
        mesh=mesh,
        compiler_params=pltpu.CompilerParams(use_tc_tiling_on_sc=False),
        scratch_types=[
            pltpu.VMEM_SHARED((NPAD, F), jnp.float32),
            pltpu.VMEM((CH,), jnp.int32),
            pltpu.VMEM((CH,), jnp.int32),
            pltpu.VMEM((CH, F), jnp.float32),
            pltpu.SemaphoreType.DMA,
        ],
    )
    return deg_call, agg_call


_BLK = 1000


def _tc1_body(x_ref, w1_ref, d0_ref, d1_ref, hp_ref, dis_ref):
    deg = d0_ref[...] + d1_ref[...] + 1.0
    dis = lax.rsqrt(deg)
    h = jnp.dot(x_ref[...], w1_ref[...],
                preferred_element_type=jnp.float32)
    hp_ref[...] = h * dis
    dis_ref[...] = jnp.broadcast_to(dis, (_BLK, F))


def _tc2_body(a0_ref, a1_ref, hp_ref, dis_ref, b1_ref, w2_ref, h2p_ref):
    dis = dis_ref[...]
    y = (a0_ref[...] + a1_ref[...] + hp_ref[...]) * dis + b1_ref[...]
    y = jnp.maximum(y, 0.0)
    h2 = jnp.dot(y, w2_ref[...], preferred_element_type=jnp.float32)
    h2p_ref[...] = h2 * dis


def _tc3_body(a0_ref, a1_ref, hp_ref, dis_ref, b2_ref, out_ref):
    z = (a0_ref[...] + a1_ref[...] + hp_ref[...]) * dis_ref[...] + b2_ref[...]
    m = jnp.max(z, axis=1, keepdims=True)
    lse = jnp.log(jnp.sum(jnp.exp(z - m), axis=1, keepdims=True)) + m
    out_ref[...] = z - lse


def _row_spec(w):
    return pl.BlockSpec((_BLK, w), lambda i: (i, 0))


def _full_spec(shape):
    return pl.BlockSpec(shape, lambda i: (0,) * len(shape))


_GRID = N_NODES // _BLK

_tc1_call = pl.pallas_call(
    _tc1_body,
    grid=(_GRID,),
    in_specs=[_row_spec(128), _full_spec((128, F)), _row_spec(1), _row_spec(1)],
    out_specs=[_row_spec(F), _row_spec(F)],
    out_shape=[jax.ShapeDtypeStruct((N_NODES, F), jnp.float32),
               jax.ShapeDtypeStruct((N_NODES, F), jnp.float32)],
)

_tc2_call = pl.pallas_call(
    _tc2_body,
    grid=(_GRID,),
    in_specs=[_row_spec(F), _row_spec(F), _row_spec(F), _row_spec(F),
              _full_spec((1, F)), _full_spec((F, F))],
    out_specs=_row_spec(F),
    out_shape=jax.ShapeDtypeStruct((N_NODES, F), jnp.float32),
)

_tc3_call = pl.pallas_call(
    _tc3_body,
    grid=(_GRID,),
    in_specs=[_row_spec(F), _row_spec(F), _row_spec(F), _row_spec(F),
              _full_spec((1, F))],
    out_specs=_row_spec(F),
    out_shape=jax.ShapeDtypeStruct((N_NODES, F), jnp.float32),
)


def kernel(x, edge_index, W1, b1, W2, b2):
    _deg_call, _agg_call = _sc_calls()
    src = edge_index[0]
    dst = edge_index[1]
    zeros1 = jnp.zeros((NPAD,), jnp.float32)
    zeros16 = jnp.zeros((NPAD, F), jnp.float32)

    degs = _deg_call(dst, zeros1)
    d0 = degs[0, :N_NODES, None]
    d1 = degs[1, :N_NODES, None]
    h1p, dis16 = _tc1_call(x, W1, d0, d1)

    agg1 = _agg_call(h1p, src, dst, zeros16)
    h2p = _tc2_call(agg1[0, :N_NODES], agg1[1, :N_NODES], h1p, dis16,
                    b1.reshape(1, F), W2)

    agg2 = _agg_call(h2p, src, dst, zeros16)
    return _tc3_call(agg2[0, :N_NODES], agg2[1, :N_NODES], h2p, dis16,
                     b2.reshape(1, F))


# speedup vs baseline: 44.6860x; 2.8022x over previous
import functools

import jax
import jax.numpy as jnp
from jax import lax
from jax.experimental import pallas as pl
from jax.experimental.pallas import tpu as pltpu
from jax.experimental.pallas import tpu_sc as plsc

N_NODES = 10000
N_EDGES = 320000
NPAD = 10240
F = 16
NUM_SC = 2
TILES = 16
WORKERS = NUM_SC * TILES
EPT = N_EDGES // WORKERS
CH = 80
CPT = EPT // CH
NB = 5
RPT = NPAD // TILES


def _deg_body(dst_hbm, zeros1_hbm, deg_out, acc_sh, didx_v, ones_v, ssem):
    c = lax.axis_index("c")
    s = lax.axis_index("s")
    wid = c * TILES + s
    for i in range(CH // 16):
        ones_v[pl.ds(i * 16, 16)] = jnp.ones((16,), jnp.float32)
    pltpu.sync_copy(dst_hbm.at[pl.ds(wid * CPT, CPT)], didx_v)
    pltpu.sync_copy(zeros1_hbm.at[pl.ds(s * RPT, RPT)],
                    acc_sh.at[pl.ds(s * RPT, RPT)])
    plsc.subcore_barrier()

    def group(o, carry):
        base = o * NB
        descs = [
            pltpu.async_copy(ones_v, acc_sh.at[didx_v.at[base + b]], ssem,
                             add=True)
            for b in range(NB)
        ]
        for d in descs:
            d.wait()
        return carry

    lax.fori_loop(0, CPT // NB, group, 0)
    plsc.subcore_barrier()
    pltpu.sync_copy(acc_sh.at[pl.ds(s * RPT, RPT)],
                    deg_out.at[c, pl.ds(s * RPT, RPT)])


def _agg_body(hp_hbm, src_hbm, dst_hbm, zeros16_hbm, agg_out,
              acc_sh, sidx_v, didx_v, rows_v, gsem, ssem):
    c = lax.axis_index("c")
    s = lax.axis_index("s")
    wid = c * TILES + s
    pltpu.sync_copy(src_hbm.at[pl.ds(wid * CPT, CPT)], sidx_v)
    pltpu.sync_copy(dst_hbm.at[pl.ds(wid * CPT, CPT)], didx_v)
    pltpu.sync_copy(zeros16_hbm.at[pl.ds(s * RPT, RPT)],
                    acc_sh.at[pl.ds(s * RPT, RPT)])
    plsc.subcore_barrier()

    def group(o, carry):
        base = o * NB
        gds = [
            pltpu.async_copy(hp_hbm.at[sidx_v.at[base + b]], rows_v.at[b],
                             gsem)
            for b in range(NB)
        ]
        for d in gds:
            d.wait()
        sds = [
            pltpu.async_copy(rows_v.at[b], acc_sh.at[didx_v.at[base + b]],
                             ssem, add=True)
            for b in range(NB)
        ]
        for d in sds:
            d.wait()
        return carry

    lax.fori_loop(0, CPT // NB, group, 0)
    plsc.subcore_barrier()
    pltpu.sync_copy(acc_sh.at[pl.ds(s * RPT, RPT)],
                    agg_out.at[c, pl.ds(s * RPT, RPT)])


@functools.cache
def _sc_calls():
    mesh = plsc.VectorSubcoreMesh(core_axis_name="c", subcore_axis_name="s",
                                  num_cores=NUM_SC, num_subcores=TILES)
    deg_call = pl.kernel(
        _deg_body,
        out_type=jax.ShapeDtypeStruct((NUM_SC, NPAD), jnp.float32),
        mesh=mesh,
        compiler_params=pltpu.CompilerParams(use_tc_tiling_on_sc=False),
        scratch_types=[
            pltpu.VMEM_SHARED((NPAD,), jnp.float32),
            pltpu.VMEM((CPT, CH), jnp.int32),
            pltpu.VMEM((CH,), jnp.float32),
            pltpu.SemaphoreType.DMA,
        ],
    )
    agg_call = pl.kernel(
        _agg_body,
        out_type=jax.ShapeDtypeStruct((NUM_SC, NPAD, F), jnp.float32),
        mesh=mesh,
        compiler_params=pltpu.CompilerParams(use_tc_tiling_on_sc=False),
        scratch_types=[
            pltpu.VMEM_SHARED((NPAD, F), jnp.float32),
            pltpu.VMEM((CPT, CH), jnp.int32),
            pltpu.VMEM((CPT, CH), jnp.int32),
            pltpu.VMEM((NB, CH, F), jnp.float32),
            pltpu.SemaphoreType.DMA,
            pltpu.SemaphoreType.DMA,
        ],
    )
    return deg_call, agg_call


_BLK = 1000


def _tc_mm_body(x_ref, w1_ref, h_ref):
    h_ref[...] = jnp.dot(x_ref[...], w1_ref[...],
                         preferred_element_type=jnp.float32)


def _tc1_body(h_ref, d0_ref, d1_ref, hp_ref, dis_ref):
    deg = d0_ref[...] + d1_ref[...] + 1.0
    dis = lax.rsqrt(deg)
    hp_ref[...] = h_ref[...] * dis
    dis_ref[...] = jnp.broadcast_to(dis, (_BLK, F))


def _tc2_body(a0_ref, a1_ref, hp_ref, dis_ref, b1_ref, w2_ref, h2p_ref):
    dis = dis_ref[...]
    y = (a0_ref[...] + a1_ref[...] + hp_ref[...]) * dis + b1_ref[...]
    y = jnp.maximum(y, 0.0)
    h2 = jnp.dot(y, w2_ref[...], preferred_element_type=jnp.float32)
    h2p_ref[...] = h2 * dis


def _tc3_body(a0_ref, a1_ref, hp_ref, dis_ref, b2_ref, out_ref):
    z = (a0_ref[...] + a1_ref[...] + hp_ref[...]) * dis_ref[...] + b2_ref[...]
    m = jnp.max(z, axis=1, keepdims=True)
    lse = jnp.log(jnp.sum(jnp.exp(z - m), axis=1, keepdims=True)) + m
    out_ref[...] = z - lse


def _row_spec(w):
    return pl.BlockSpec((_BLK, w), lambda i: (i, 0))


def _full_spec(shape):
    return pl.BlockSpec(shape, lambda i: (0,) * len(shape))


_GRID = N_NODES // _BLK

_tc_mm_call = pl.pallas_call(
    _tc_mm_body,
    grid=(_GRID,),
    in_specs=[_row_spec(128), _full_spec((128, F))],
    out_specs=_row_spec(F),
    out_shape=jax.ShapeDtypeStruct((N_NODES, F), jnp.float32),
)

_tc1_call = pl.pallas_call(
    _tc1_body,
    grid=(_GRID,),
    in_specs=[_row_spec(F), _row_spec(1), _row_spec(1)],
    out_specs=[_row_spec(F), _row_spec(F)],
    out_shape=[jax.ShapeDtypeStruct((N_NODES, F), jnp.float32),
               jax.ShapeDtypeStruct((N_NODES, F), jnp.float32)],
)

_tc2_call = pl.pallas_call(
    _tc2_body,
    grid=(_GRID,),
    in_specs=[_row_spec(F), _row_spec(F), _row_spec(F), _row_spec(F),
              _full_spec((1, F)), _full_spec((F, F))],
    out_specs=_row_spec(F),
    out_shape=jax.ShapeDtypeStruct((N_NODES, F), jnp.float32),
)

_tc3_call = pl.pallas_call(
    _tc3_body,
    grid=(_GRID,),
    in_specs=[_row_spec(F), _row_spec(F), _row_spec(F), _row_spec(F),
              _full_spec((1, F))],
    out_specs=_row_spec(F),
    out_shape=jax.ShapeDtypeStruct((N_NODES, F), jnp.float32),
)


def kernel(x, edge_index, W1, b1, W2, b2):
    _deg_call, _agg_call = _sc_calls()
    src = edge_index[0].reshape(WORKERS * CPT, CH)
    dst = edge_index[1].reshape(WORKERS * CPT, CH)
    zeros1 = jnp.zeros((NPAD,), jnp.float32)
    zeros16 = jnp.zeros((NPAD, F), jnp.float32)

    h1 = _tc_mm_call(x, W1)
    degs = _deg_call(dst, zeros1)
    d0 = degs[0, :N_NODES, None]
    d1 = degs[1, :N_NODES, None]
    h1p, dis16 = _tc1_call(h1, d0, d1)

    agg1 = _agg_call(h1p, src, dst, zeros16)
    h2p = _tc2_call(agg1[0, :N_NODES], agg1[1, :N_NODES], h1p, dis16,
                    b1.reshape(1, F), W2)

    agg2 = _agg_call(h2p, src, dst, zeros16)
    return _tc3_call(agg2[0, :N_NODES], agg2[1, :N_NODES], h2p, dis16,
                     b2.reshape(1, F))


# speedup vs baseline: 55.5633x; 1.2434x over previous
import functools

import jax
import jax.numpy as jnp
from jax import lax
from jax.experimental import pallas as pl
from jax.experimental.pallas import tpu as pltpu
from jax.experimental.pallas import tpu_sc as plsc

N_NODES = 10000
N_EDGES = 320000
NPAD = 10240
F = 16
NUM_SC = 2
TILES = 16
WORKERS = NUM_SC * TILES
EPT = N_EDGES // WORKERS
CH = 80
CPT = EPT // CH
NB = 5
RPT = NPAD // TILES


def _deg_body(dst_hbm, zeros1_hbm, deg_out, acc_sh, didx_v, ones_v, ssem):
    c = lax.axis_index("c")
    s = lax.axis_index("s")
    wid = c * TILES + s
    for i in range(CH // 16):
        ones_v[pl.ds(i * 16, 16)] = jnp.ones((16,), jnp.float32)
    pltpu.sync_copy(dst_hbm.at[pl.ds(wid * CPT, CPT)], didx_v)
    pltpu.sync_copy(zeros1_hbm.at[pl.ds(s * RPT, RPT)],
                    acc_sh.at[pl.ds(s * RPT, RPT)])
    plsc.subcore_barrier()

    def group(o, carry):
        base = o * NB
        descs = [
            pltpu.async_copy(ones_v, acc_sh.at[didx_v.at[base + b]], ssem,
                             add=True)
            for b in range(NB)
        ]
        for d in descs:
            d.wait()
        return carry

    lax.fori_loop(0, CPT // NB, group, 0)
    plsc.subcore_barrier()
    pltpu.sync_copy(acc_sh.at[pl.ds(s * RPT, RPT)],
                    deg_out.at[c, pl.ds(s * RPT, RPT)])


def _agg_body(hp_hbm, src_hbm, dst_hbm, zeros16_hbm, agg_out,
              acc_sh, sidx_v, didx_v, rows_v, gsem, ssem):
    c = lax.axis_index("c")
    s = lax.axis_index("s")
    wid = c * TILES + s
    pltpu.sync_copy(src_hbm.at[pl.ds(wid * CPT, CPT)], sidx_v)
    pltpu.sync_copy(dst_hbm.at[pl.ds(wid * CPT, CPT)], didx_v)
    pltpu.sync_copy(zeros16_hbm.at[pl.ds(s * RPT, RPT)],
                    acc_sh.at[pl.ds(s * RPT, RPT)])
    plsc.subcore_barrier()

    ngroups = CPT // NB

    def _drain(src, dst, sem):
        pltpu.make_async_copy(src, dst, sem).wait()

    for b in range(NB):
        pltpu.async_copy(hp_hbm.at[sidx_v.at[b]], rows_v.at[b], gsem)

    def group(o, carry):
        half = lax.rem(o, 2)
        buf0 = half * NB
        nbuf0 = (1 - half) * NB

        @pl.when(o > 0)
        def _():
            for b in range(NB):
                _drain(rows_v.at[0], acc_sh.at[pl.ds(0, CH)], ssem)

        @pl.when(o + 1 < ngroups)
        def _():
            for b in range(NB):
                pltpu.async_copy(hp_hbm.at[sidx_v.at[(o + 1) * NB + b]],
                                 rows_v.at[nbuf0 + b], gsem)

        for b in range(NB):
            _drain(hp_hbm.at[pl.ds(0, CH)], rows_v.at[0], gsem)
        for b in range(NB):
            pltpu.async_copy(rows_v.at[buf0 + b],
                             acc_sh.at[didx_v.at[o * NB + b]], ssem, add=True)
        return carry

    lax.fori_loop(0, ngroups, group, 0)
    for b in range(NB):
        _drain(rows_v.at[0], acc_sh.at[pl.ds(0, CH)], ssem)
    plsc.subcore_barrier()
    pltpu.sync_copy(acc_sh.at[pl.ds(s * RPT, RPT)],
                    agg_out.at[c, pl.ds(s * RPT, RPT)])


@functools.cache
def _sc_calls():
    mesh = plsc.VectorSubcoreMesh(core_axis_name="c", subcore_axis_name="s",
                                  num_cores=NUM_SC, num_subcores=TILES)
    deg_call = pl.kernel(
        _deg_body,
        out_type=jax.ShapeDtypeStruct((NUM_SC, NPAD), jnp.float32),
        mesh=mesh,
        compiler_params=pltpu.CompilerParams(use_tc_tiling_on_sc=False),
        scratch_types=[
            pltpu.VMEM_SHARED((NPAD,), jnp.float32),
            pltpu.VMEM((CPT, CH), jnp.int32),
            pltpu.VMEM((CH,), jnp.float32),
            pltpu.SemaphoreType.DMA,
        ],
    )
    agg_call = pl.kernel(
        _agg_body,
        out_type=jax.ShapeDtypeStruct((NUM_SC, NPAD, F), jnp.float32),
        mesh=mesh,
        compiler_params=pltpu.CompilerParams(use_tc_tiling_on_sc=False),
        scratch_types=[
            pltpu.VMEM_SHARED((NPAD, F), jnp.float32),
            pltpu.VMEM((CPT, CH), jnp.int32),
            pltpu.VMEM((CPT, CH), jnp.int32),
            pltpu.VMEM((2 * NB, CH, F), jnp.float32),
            pltpu.SemaphoreType.DMA,
            pltpu.SemaphoreType.DMA,
        ],
    )
    return deg_call, agg_call


_BLK = 1000


def _tc_mm_body(x_ref, w1_ref, h_ref):
    h_ref[...] = jnp.dot(x_ref[...], w1_ref[...],
                         preferred_element_type=jnp.float32)


def _tc1_body(h_ref, d0_ref, d1_ref, hp_ref, dis_ref):
    deg = d0_ref[...] + d1_ref[...] + 1.0
    dis = lax.rsqrt(deg)
    hp_ref[...] = h_ref[...] * dis
    dis_ref[...] = jnp.broadcast_to(dis, (_BLK, F))


def _tc2_body(a0_ref, a1_ref, hp_ref, dis_ref, b1_ref, w2_ref, h2p_ref):
    dis = dis_ref[...]
    y = (a0_ref[0] + a1_ref[0] + hp_ref[...]) * dis + b1_ref[...]
    y = jnp.maximum(y, 0.0)
    h2 = jnp.dot(y, w2_ref[...], preferred_element_type=jnp.float32)
    h2p_ref[...] = h2 * dis


def _tc3_body(a0_ref, a1_ref, hp_ref, dis_ref, b2_ref, out_ref):
    z = (a0_ref[0] + a1_ref[0] + hp_ref[...]) * dis_ref[...] + b2_ref[...]
    m = jnp.max(z, axis=1, keepdims=True)
    lse = jnp.log(jnp.sum(jnp.exp(z - m), axis=1, keepdims=True)) + m
    out_ref[...] = z - lse


def _row_spec(w):
    return pl.BlockSpec((_BLK, w), lambda i: (i, 0))


def _full_spec(shape):
    return pl.BlockSpec(shape, lambda i: (0,) * len(shape))


_GRID = N_NODES // _BLK

_tc_mm_call = pl.pallas_call(
    _tc_mm_body,
    grid=(_GRID,),
    in_specs=[_row_spec(128), _full_spec((128, F))],
    out_specs=_row_spec(F),
    out_shape=jax.ShapeDtypeStruct((N_NODES, F), jnp.float32),
)

_tc1_call = pl.pallas_call(
    _tc1_body,
    grid=(_GRID,),
    in_specs=[_row_spec(F), _row_spec(1), _row_spec(1)],
    out_specs=[_row_spec(F), _row_spec(F)],
    out_shape=[jax.ShapeDtypeStruct((N_NODES, F), jnp.float32),
               jax.ShapeDtypeStruct((N_NODES, F), jnp.float32)],
)

_agg0_spec = pl.BlockSpec((1, _BLK, F), lambda i: (0, i, 0))
_agg1_spec = pl.BlockSpec((1, _BLK, F), lambda i: (1, i, 0))

_tc2_call = pl.pallas_call(
    _tc2_body,
    grid=(_GRID,),
    in_specs=[_agg0_spec, _agg1_spec, _row_spec(F), _row_spec(F),
              _full_spec((1, F)), _full_spec((F, F))],
    out_specs=_row_spec(F),
    out_shape=jax.ShapeDtypeStruct((N_NODES, F), jnp.float32),
)

_tc3_call = pl.pallas_call(
    _tc3_body,
    grid=(_GRID,),
    in_specs=[_agg0_spec, _agg1_spec, _row_spec(F), _row_spec(F),
              _full_spec((1, F))],
    out_specs=_row_spec(F),
    out_shape=jax.ShapeDtypeStruct((N_NODES, F), jnp.float32),
)


def kernel(x, edge_index, W1, b1, W2, b2):
    _deg_call, _agg_call = _sc_calls()
    src = edge_index[0].reshape(WORKERS * CPT, CH)
    dst = edge_index[1].reshape(WORKERS * CPT, CH)
    zeros1 = jnp.zeros((NPAD,), jnp.float32)
    zeros16 = jnp.zeros((NPAD, F), jnp.float32)

    h1 = _tc_mm_call(x, W1)
    degs = _deg_call(dst, zeros1)
    d0 = degs[0, :N_NODES, None]
    d1 = degs[1, :N_NODES, None]
    h1p, dis16 = _tc1_call(h1, d0, d1)

    agg1 = _agg_call(h1p, src, dst, zeros16)
    h2p = _tc2_call(agg1, agg1, h1p, dis16, b1.reshape(1, F), W2)

    agg2 = _agg_call(h2p, src, dst, zeros16)
    return _tc3_call(agg2, agg2, h2p, dis16, b2.reshape(1, F))


# speedup vs baseline: 59.3453x; 1.0681x over previous
import functools

import jax
import jax.numpy as jnp
from jax import lax
from jax.experimental import pallas as pl
from jax.experimental.pallas import tpu as pltpu
from jax.experimental.pallas import tpu_sc as plsc

N_NODES = 10000
N_EDGES = 320000
NPAD = 10240
F = 16
NUM_SC = 2
TILES = 16
WORKERS = NUM_SC * TILES
EPT = N_EDGES // WORKERS
CH = 80
CPT = EPT // CH
DCPT = N_EDGES // TILES // CH
NB = 5
RPT = NPAD // TILES
VG = RPT // 16
FULL_T = 15
TAIL_ROWS = N_NODES - FULL_T * RPT
MAGIC = 0x5F3759DF


def _rsqrt16(x):
    i = lax.bitcast_convert_type(x, jnp.int32)
    i = jnp.full((16,), MAGIC, jnp.int32) - lax.shift_right_logical(
        i, jnp.full((16,), 1, jnp.int32))
    y = lax.bitcast_convert_type(i, jnp.float32)
    xh = x * 0.5
    for _ in range(3):
        y = y * (1.5 - xh * y * y)
    return y


def _sc1_body(h1_hbm, dsta_hbm, src_hbm, dst_hbm, zeros1_hbm, zeros16_hbm,
              dis_out, agg_out,
              acc_sh, hp_sh, deg_sh,
              didxd_v, sidx_v, didx_v, deg_v, dis_v, rowbuf, rows_v, ones_v,
              gsem, ssem):
    c = lax.axis_index("c")
    s = lax.axis_index("s")
    wid = c * TILES + s

    for i in range(CH // 16):
        ones_v[pl.ds(i * 16, 16)] = jnp.ones((16,), jnp.float32)
    pltpu.sync_copy(dsta_hbm.at[pl.ds(s * DCPT, DCPT)], didxd_v)
    pltpu.sync_copy(src_hbm.at[pl.ds(wid * CPT, CPT)], sidx_v)
    pltpu.sync_copy(dst_hbm.at[pl.ds(wid * CPT, CPT)], didx_v)
    pltpu.sync_copy(zeros1_hbm.at[pl.ds(s * RPT, RPT)],
                    deg_sh.at[pl.ds(s * RPT, RPT)])
    pltpu.sync_copy(zeros16_hbm.at[pl.ds(s * RPT, RPT)],
                    acc_sh.at[pl.ds(s * RPT, RPT)])
    @pl.when(s < FULL_T)
    def _():
        pltpu.sync_copy(h1_hbm.at[pl.ds(s * RPT, RPT)], rowbuf)

    @pl.when(s == FULL_T)
    def _():
        pltpu.sync_copy(h1_hbm.at[pl.ds(FULL_T * RPT, TAIL_ROWS)],
                        rowbuf.at[pl.ds(0, TAIL_ROWS)])

    plsc.subcore_barrier()

    def dgroup(o, carry):
        base = o * NB
        descs = [
            pltpu.async_copy(ones_v, deg_sh.at[didxd_v.at[base + b]], ssem,
                             add=True)
            for b in range(NB)
        ]
        for d in descs:
            d.wait()
        return carry

    lax.fori_loop(0, DCPT // NB, dgroup, 0)
    plsc.subcore_barrier()

    pltpu.sync_copy(deg_sh.at[pl.ds(s * RPT, RPT)], deg_v)

    def disg(g, carry):
        d = deg_v[pl.ds(g * 16, 16)] + 1.0
        dis_v[pl.ds(g * 16, 16)] = _rsqrt16(d)
        return carry

    lax.fori_loop(0, VG, disg, 0)

    for g in range(VG):
        disv = dis_v[pl.ds(g * 16, 16)]
        for k in range(16):
            r = g * 16 + k
            rowbuf[r] = rowbuf[r] * jnp.broadcast_to(disv[k], (16,))

    @pl.when(c == 0)
    def _():
        pltpu.sync_copy(dis_v, dis_out.at[pl.ds(s * RPT, RPT)])

    @pl.when(s < FULL_T)
    def _():
        pltpu.sync_copy(rowbuf, hp_sh.at[pl.ds(s * RPT, RPT)])

    @pl.when(s == FULL_T)
    def _():
        pltpu.sync_copy(rowbuf.at[pl.ds(0, TAIL_ROWS)],
                        hp_sh.at[pl.ds(FULL_T * RPT, TAIL_ROWS)])

    plsc.subcore_barrier()

    ngroups = CPT // NB

    def _drain(src, dst, sem):
        pltpu.make_async_copy(src, dst, sem).wait()

    for b in range(NB):
        pltpu.async_copy(hp_sh.at[sidx_v.at[b]], rows_v.at[b], gsem)

    def group(o, carry):
        half = lax.rem(o, 2)
        buf0 = half * NB
        nbuf0 = (1 - half) * NB

        @pl.when(o > 0)
        def _():
            for b in range(NB):
                _drain(rows_v.at[0], acc_sh.at[pl.ds(0, CH)], ssem)

        @pl.when(o + 1 < ngroups)
        def _():
            for b in range(NB):
                pltpu.async_copy(hp_sh.at[sidx_v.at[(o + 1) * NB + b]],
                                 rows_v.at[nbuf0 + b], gsem)

        for b in range(NB):
            _drain(h1_hbm.at[pl.ds(0, CH)], rows_v.at[0], gsem)
        for b in range(NB):
            pltpu.async_copy(rows_v.at[buf0 + b],
                             acc_sh.at[didx_v.at[o * NB + b]], ssem, add=True)
        return carry

    lax.fori_loop(0, ngroups, group, 0)
    for b in range(NB):
        _drain(rows_v.at[0], acc_sh.at[pl.ds(0, CH)], ssem)
    plsc.subcore_barrier()
    pltpu.sync_copy(acc_sh.at[pl.ds(s * RPT, RPT)],
                    agg_out.at[c, pl.ds(s * RPT, RPT)])


def _sc2_body(hp_hbm, src_hbm, dst_hbm, zeros16_hbm, agg_out,
              acc_sh, hp_sh, sidx_v, didx_v, rows_v, gsem, ssem):
    c = lax.axis_index("c")
    s = lax.axis_index("s")
    wid = c * TILES + s
    pltpu.sync_copy(src_hbm.at[pl.ds(wid * CPT, CPT)], sidx_v)
    pltpu.sync_copy(dst_hbm.at[pl.ds(wid * CPT, CPT)], didx_v)
    pltpu.sync_copy(zeros16_hbm.at[pl.ds(s * RPT, RPT)],
                    acc_sh.at[pl.ds(s * RPT, RPT)])
    @pl.when(s < FULL_T)
    def _():
        pltpu.sync_copy(hp_hbm.at[pl.ds(s * RPT, RPT)],
                        hp_sh.at[pl.ds(s * RPT, RPT)])

    @pl.when(s == FULL_T)
    def _():
        pltpu.sync_copy(hp_hbm.at[pl.ds(FULL_T * RPT, TAIL_ROWS)],
                        hp_sh.at[pl.ds(FULL_T * RPT, TAIL_ROWS)])

    plsc.subcore_barrier()

    ngroups = CPT // NB

    def _drain(src, dst, sem):
        pltpu.make_async_copy(src, dst, sem).wait()

    for b in range(NB):
        pltpu.async_copy(hp_sh.at[sidx_v.at[b]], rows_v.at[b], gsem)

    def group(o, carry):
        half = lax.rem(o, 2)
        buf0 = half * NB
        nbuf0 = (1 - half) * NB

        @pl.when(o > 0)
        def _():
            for b in range(NB):
                _drain(rows_v.at[0], acc_sh.at[pl.ds(0, CH)], ssem)

        @pl.when(o + 1 < ngroups)
        def _():
            for b in range(NB):
                pltpu.async_copy(hp_sh.at[sidx_v.at[(o + 1) * NB + b]],
                                 rows_v.at[nbuf0 + b], gsem)

        for b in range(NB):
            _drain(hp_hbm.at[pl.ds(0, CH)], rows_v.at[0], gsem)
        for b in range(NB):
            pltpu.async_copy(rows_v.at[buf0 + b],
                             acc_sh.at[didx_v.at[o * NB + b]], ssem, add=True)
        return carry

    lax.fori_loop(0, ngroups, group, 0)
    for b in range(NB):
        _drain(rows_v.at[0], acc_sh.at[pl.ds(0, CH)], ssem)
    plsc.subcore_barrier()
    pltpu.sync_copy(acc_sh.at[pl.ds(s * RPT, RPT)],
                    agg_out.at[c, pl.ds(s * RPT, RPT)])


@functools.cache
def _sc_calls():
    mesh = plsc.VectorSubcoreMesh(core_axis_name="c", subcore_axis_name="s",
                                  num_cores=NUM_SC, num_subcores=TILES)
    sc1_call = pl.kernel(
        _sc1_body,
        out_type=[jax.ShapeDtypeStruct((NPAD,), jnp.float32),
                  jax.ShapeDtypeStruct((NUM_SC, NPAD, F), jnp.float32)],
        mesh=mesh,
        compiler_params=pltpu.CompilerParams(use_tc_tiling_on_sc=False),
        scratch_types=[
            pltpu.VMEM_SHARED((NPAD, F), jnp.float32),
            pltpu.VMEM_SHARED((NPAD, F), jnp.float32),
            pltpu.VMEM_SHARED((NPAD,), jnp.float32),
            pltpu.VMEM((DCPT, CH), jnp.int32),
            pltpu.VMEM((CPT, CH), jnp.int32),
            pltpu.VMEM((CPT, CH), jnp.int32),
            pltpu.VMEM((RPT,), jnp.float32),
            pltpu.VMEM((RPT,), jnp.float32),
            pltpu.VMEM((RPT, F), jnp.float32),
            pltpu.VMEM((2 * NB, CH, F), jnp.float32),
            pltpu.VMEM((CH,), jnp.float32),
            pltpu.SemaphoreType.DMA,
            pltpu.SemaphoreType.DMA,
        ],
    )
    sc2_call = pl.kernel(
        _sc2_body,
        out_type=jax.ShapeDtypeStruct((NUM_SC, NPAD, F), jnp.float32),
        mesh=mesh,
        compiler_params=pltpu.CompilerParams(use_tc_tiling_on_sc=False),
        scratch_types=[
            pltpu.VMEM_SHARED((NPAD, F), jnp.float32),
            pltpu.VMEM_SHARED((NPAD, F), jnp.float32),
            pltpu.VMEM((CPT, CH), jnp.int32),
            pltpu.VMEM((CPT, CH), jnp.int32),
            pltpu.VMEM((2 * NB, CH, F), jnp.float32),
            pltpu.SemaphoreType.DMA,
            pltpu.SemaphoreType.DMA,
        ],
    )
    return sc1_call, sc2_call


_BLK = 1000


def _tc_mm_body(x_ref, w1_ref, h_ref):
    h_ref[...] = jnp.dot(x_ref[...], w1_ref[...],
                         preferred_element_type=jnp.float32)


def _tc2_body(a0_ref, a1_ref, h1_ref, dis_ref, b1_ref, w2_ref, h2p_ref):
    dis = dis_ref[...]
    y = (a0_ref[0] + a1_ref[0]) * dis + h1_ref[...] * (dis * dis) + b1_ref[...]
    y = jnp.maximum(y, 0.0)
    h2 = jnp.dot(y, w2_ref[...], preferred_element_type=jnp.float32)
    h2p_ref[...] = h2 * dis


def _tc3_body(a0_ref, a1_ref, hp_ref, dis_ref, b2_ref, out_ref):
    z = (a0_ref[0] + a1_ref[0] + hp_ref[...]) * dis_ref[...] + b2_ref[...]
    m = jnp.max(z, axis=1, keepdims=True)
    lse = jnp.log(jnp.sum(jnp.exp(z - m), axis=1, keepdims=True)) + m
    out_ref[...] = z - lse


def _row_spec(w):
    return pl.BlockSpec((_BLK, w), lambda i: (i, 0))


def _full_spec(shape):
    return pl.BlockSpec(shape, lambda i: (0,) * len(shape))


_GRID = N_NODES // _BLK

_tc_mm_call = pl.pallas_call(
    _tc_mm_body,
    grid=(_GRID,),
    in_specs=[_row_spec(128), _full_spec((128, F))],
    out_specs=_row_spec(F),
    out_shape=jax.ShapeDtypeStruct((N_NODES, F), jnp.float32),
)

_agg0_spec = pl.BlockSpec((1, _BLK, F), lambda i: (0, i, 0))
_agg1_spec = pl.BlockSpec((1, _BLK, F), lambda i: (1, i, 0))

_tc2_call = pl.pallas_call(
    _tc2_body,
    grid=(_GRID,),
    in_specs=[_agg0_spec, _agg1_spec, _row_spec(F), _row_spec(1),
              _full_spec((1, F)), _full_spec((F, F))],
    out_specs=_row_spec(F),
    out_shape=jax.ShapeDtypeStruct((N_NODES, F), jnp.float32),
)

_tc3_call = pl.pallas_call(
    _tc3_body,
    grid=(_GRID,),
    in_specs=[_agg0_spec, _agg1_spec, _row_spec(F), _row_spec(1),
              _full_spec((1, F))],
    out_specs=_row_spec(F),
    out_shape=jax.ShapeDtypeStruct((N_NODES, F), jnp.float32),
)


def kernel(x, edge_index, W1, b1, W2, b2):
    sc1_call, sc2_call = _sc_calls()
    src = edge_index[0].reshape(WORKERS * CPT, CH)
    dst = edge_index[1].reshape(WORKERS * CPT, CH)
    zeros1 = jnp.zeros((NPAD,), jnp.float32)
    zeros16 = jnp.zeros((NPAD, F), jnp.float32)

    h1 = _tc_mm_call(x, W1)
    dis, agg1 = sc1_call(h1, dst, src, dst, zeros1, zeros16)
    diss = dis[:N_NODES, None]

    h2p = _tc2_call(agg1, agg1, h1, diss, b1.reshape(1, F), W2)
    agg2 = sc2_call(h2p, src, dst, zeros16)
    return _tc3_call(agg2, agg2, h2p, diss, b2.reshape(1, F))


# speedup vs baseline: 61.1839x; 1.0310x over previous
import functools

import jax
import jax.numpy as jnp
from jax import lax
from jax.experimental import pallas as pl
from jax.experimental.pallas import tpu as pltpu
from jax.experimental.pallas import tpu_sc as plsc

N_NODES = 10000
N_EDGES = 320000
NPAD = 10240
F = 16
NUM_SC = 2
TILES = 16
WORKERS = NUM_SC * TILES
EPT = N_EDGES // WORKERS
CH = 80
CPT = EPT // CH
DCPT = N_EDGES // TILES // CH
NB = 5
RPT = NPAD // TILES
VG = RPT // 16
FULL_T = 15
TAIL_ROWS = N_NODES - FULL_T * RPT
MAGIC = 0x5F3759DF


def _rsqrt16(x):
    i = lax.bitcast_convert_type(x, jnp.int32)
    i = jnp.full((16,), MAGIC, jnp.int32) - lax.shift_right_logical(
        i, jnp.full((16,), 1, jnp.int32))
    y = lax.bitcast_convert_type(i, jnp.float32)
    xh = x * 0.5
    for _ in range(3):
        y = y * (1.5 - xh * y * y)
    return y


def _sc1_body(h1_hbm, dsta_hbm, src_hbm, dst_hbm, zeros1_hbm, zeros16_hbm,
              dis_out, agg_out,
              acc_sh, hp_sh, deg_sh,
              didxd_v, sidx_v, didx_v, deg_v, dis_v, rowbuf, rows_v, ones_v,
              gsem, ssem):
    c = lax.axis_index("c")
    s = lax.axis_index("s")
    wid = c * TILES + s

    for i in range(CH // 16):
        ones_v[pl.ds(i * 16, 16)] = jnp.ones((16,), jnp.float32)
    pltpu.sync_copy(dsta_hbm.at[pl.ds(s * DCPT, DCPT)], didxd_v)
    pltpu.sync_copy(src_hbm.at[pl.ds(wid * CPT, CPT)], sidx_v)
    pltpu.sync_copy(dst_hbm.at[pl.ds(wid * CPT, CPT)], didx_v)
    pltpu.sync_copy(zeros1_hbm.at[pl.ds(s * RPT, RPT)],
                    deg_sh.at[pl.ds(s * RPT, RPT)])
    pltpu.sync_copy(zeros16_hbm.at[pl.ds(s * RPT, RPT)],
                    acc_sh.at[pl.ds(s * RPT, RPT)])
    @pl.when(s < FULL_T)
    def _():
        pltpu.sync_copy(h1_hbm.at[pl.ds(s * RPT, RPT)], rowbuf)

    @pl.when(s == FULL_T)
    def _():
        pltpu.sync_copy(h1_hbm.at[pl.ds(FULL_T * RPT, TAIL_ROWS)],
                        rowbuf.at[pl.ds(0, TAIL_ROWS)])

    plsc.subcore_barrier()

    def dgroup(o, carry):
        base = o * NB
        descs = [
            pltpu.async_copy(ones_v, deg_sh.at[didxd_v.at[base + b]], ssem,
                             add=True)
            for b in range(NB)
        ]
        for d in descs:
            d.wait()
        return carry

    lax.fori_loop(0, DCPT // NB, dgroup, 0)
    plsc.subcore_barrier()

    pltpu.sync_copy(deg_sh.at[pl.ds(s * RPT, RPT)], deg_v)

    def disg(g, carry):
        d = deg_v[pl.ds(g * 16, 16)] + 1.0
        dis_v[pl.ds(g * 16, 16)] = _rsqrt16(d)
        return carry

    lax.fori_loop(0, VG, disg, 0)

    for g in range(VG):
        disv = dis_v[pl.ds(g * 16, 16)]
        for k in range(16):
            r = g * 16 + k
            rowbuf[r] = rowbuf[r] * jnp.broadcast_to(disv[k], (16,))

    @pl.when(c == 0)
    def _():
        pltpu.sync_copy(dis_v, dis_out.at[pl.ds(s * RPT, RPT)])

    @pl.when(s < FULL_T)
    def _():
        pltpu.sync_copy(rowbuf, hp_sh.at[pl.ds(s * RPT, RPT)])

    @pl.when(s == FULL_T)
    def _():
        pltpu.sync_copy(rowbuf.at[pl.ds(0, TAIL_ROWS)],
                        hp_sh.at[pl.ds(FULL_T * RPT, TAIL_ROWS)])

    plsc.subcore_barrier()

    ngroups = CPT // NB

    def _drain(src, dst, sem):
        pltpu.make_async_copy(src, dst, sem).wait()

    for b in range(NB):
        pltpu.async_copy(hp_sh.at[sidx_v.at[b]], rows_v.at[b], gsem)

    def group(o, carry):
        half = lax.rem(o, 2)
        buf0 = half * NB
        nbuf0 = (1 - half) * NB

        @pl.when(o > 0)
        def _():
            for b in range(NB):
                _drain(rows_v.at[0], acc_sh.at[pl.ds(0, CH)], ssem)

        @pl.when(o + 1 < ngroups)
        def _():
            for b in range(NB):
                pltpu.async_copy(hp_sh.at[sidx_v.at[(o + 1) * NB + b]],
                                 rows_v.at[nbuf0 + b], gsem)

        for b in range(NB):
            _drain(h1_hbm.at[pl.ds(0, CH)], rows_v.at[0], gsem)
        for b in range(NB):
            pltpu.async_copy(rows_v.at[buf0 + b],
                             acc_sh.at[didx_v.at[o * NB + b]], ssem, add=True)
        return carry

    lax.fori_loop(0, ngroups, group, 0)
    for b in range(NB):
        _drain(rows_v.at[0], acc_sh.at[pl.ds(0, CH)], ssem)
    plsc.subcore_barrier()
    pltpu.sync_copy(acc_sh.at[pl.ds(s * RPT, RPT)],
                    agg_out.at[c, pl.ds(s * RPT, RPT)])


def _sc2_body(hp_hbm, src_hbm, dst_hbm, zeros16_hbm, agg_out,
              acc_sh, hp_sh, sidx_v, didx_v, rows_v, gsem, ssem):
    c = lax.axis_index("c")
    s = lax.axis_index("s")
    wid = c * TILES + s
    pltpu.sync_copy(src_hbm.at[pl.ds(wid * CPT, CPT)], sidx_v)
    pltpu.sync_copy(dst_hbm.at[pl.ds(wid * CPT, CPT)], didx_v)
    pltpu.sync_copy(zeros16_hbm.at[pl.ds(s * RPT, RPT)],
                    acc_sh.at[pl.ds(s * RPT, RPT)])
    @pl.when(s < FULL_T)
    def _():
        pltpu.sync_copy(hp_hbm.at[pl.ds(s * RPT, RPT)],
                        hp_sh.at[pl.ds(s * RPT, RPT)])

    @pl.when(s == FULL_T)
    def _():
        pltpu.sync_copy(hp_hbm.at[pl.ds(FULL_T * RPT, TAIL_ROWS)],
                        hp_sh.at[pl.ds(FULL_T * RPT, TAIL_ROWS)])

    plsc.subcore_barrier()

    ngroups = CPT // NB

    def _drain(src, dst, sem):
        pltpu.make_async_copy(src, dst, sem).wait()

    for b in range(NB):
        pltpu.async_copy(hp_sh.at[sidx_v.at[b]], rows_v.at[b], gsem)

    def group(o, carry):
        half = lax.rem(o, 2)
        buf0 = half * NB
        nbuf0 = (1 - half) * NB

        @pl.when(o > 0)
        def _():
            for b in range(NB):
                _drain(rows_v.at[0], acc_sh.at[pl.ds(0, CH)], ssem)

        @pl.when(o + 1 < ngroups)
        def _():
            for b in range(NB):
                pltpu.async_copy(hp_sh.at[sidx_v.at[(o + 1) * NB + b]],
                                 rows_v.at[nbuf0 + b], gsem)

        for b in range(NB):
            _drain(hp_hbm.at[pl.ds(0, CH)], rows_v.at[0], gsem)
        for b in range(NB):
            pltpu.async_copy(rows_v.at[buf0 + b],
                             acc_sh.at[didx_v.at[o * NB + b]], ssem, add=True)
        return carry

    lax.fori_loop(0, ngroups, group, 0)
    for b in range(NB):
        _drain(rows_v.at[0], acc_sh.at[pl.ds(0, CH)], ssem)
    plsc.subcore_barrier()
    pltpu.sync_copy(acc_sh.at[pl.ds(s * RPT, RPT)],
                    agg_out.at[c, pl.ds(s * RPT, RPT)])


@functools.cache
def _sc_calls():
    mesh = plsc.VectorSubcoreMesh(core_axis_name="c", subcore_axis_name="s",
                                  num_cores=NUM_SC, num_subcores=TILES)
    sc1_call = pl.kernel(
        _sc1_body,
        out_type=[jax.ShapeDtypeStruct((NPAD,), jnp.float32),
                  jax.ShapeDtypeStruct((NUM_SC, NPAD, F), jnp.float32)],
        mesh=mesh,
        compiler_params=pltpu.CompilerParams(use_tc_tiling_on_sc=False),
        scratch_types=[
            pltpu.VMEM_SHARED((NPAD, F), jnp.float32),
            pltpu.VMEM_SHARED((NPAD, F), jnp.float32),
            pltpu.VMEM_SHARED((NPAD,), jnp.float32),
            pltpu.VMEM((DCPT, CH), jnp.int32),
            pltpu.VMEM((CPT, CH), jnp.int32),
            pltpu.VMEM((CPT, CH), jnp.int32),
            pltpu.VMEM((RPT,), jnp.float32),
            pltpu.VMEM((RPT,), jnp.float32),
            pltpu.VMEM((RPT, F), jnp.float32),
            pltpu.VMEM((2 * NB, CH, F), jnp.float32),
            pltpu.VMEM((CH,), jnp.float32),
            pltpu.SemaphoreType.DMA,
            pltpu.SemaphoreType.DMA,
        ],
    )
    sc2_call = pl.kernel(
        _sc2_body,
        out_type=jax.ShapeDtypeStruct((NUM_SC, NPAD, F), jnp.float32),
        mesh=mesh,
        compiler_params=pltpu.CompilerParams(use_tc_tiling_on_sc=False),
        scratch_types=[
            pltpu.VMEM_SHARED((NPAD, F), jnp.float32),
            pltpu.VMEM_SHARED((NPAD, F), jnp.float32),
            pltpu.VMEM((CPT, CH), jnp.int32),
            pltpu.VMEM((CPT, CH), jnp.int32),
            pltpu.VMEM((2 * NB, CH, F), jnp.float32),
            pltpu.SemaphoreType.DMA,
            pltpu.SemaphoreType.DMA,
        ],
    )
    return sc1_call, sc2_call


def _tc_mm_body(x_ref, w1_ref, h_ref):
    h_ref[...] = jnp.dot(x_ref[...], w1_ref[...],
                         preferred_element_type=jnp.float32)


def _tc2_body(agg_ref, h1_ref, dis_ref, b1_ref, w2_ref, h2p_ref):
    dis = dis_ref[...]
    agg = agg_ref[0, :N_NODES] + agg_ref[1, :N_NODES]
    y = agg * dis + h1_ref[...] * (dis * dis) + b1_ref[...]
    y = jnp.maximum(y, 0.0)
    h2 = jnp.dot(y, w2_ref[...], preferred_element_type=jnp.float32)
    h2p_ref[...] = h2 * dis


def _tc3_body(agg_ref, hp_ref, dis_ref, b2_ref, out_ref):
    agg = agg_ref[0, :N_NODES] + agg_ref[1, :N_NODES]
    z = (agg + hp_ref[...]) * dis_ref[...] + b2_ref[...]
    m = jnp.max(z, axis=1, keepdims=True)
    lse = jnp.log(jnp.sum(jnp.exp(z - m), axis=1, keepdims=True)) + m
    out_ref[...] = z - lse


_tc_mm_call = pl.pallas_call(
    _tc_mm_body,
    out_shape=jax.ShapeDtypeStruct((N_NODES, F), jnp.float32),
)

_tc2_call = pl.pallas_call(
    _tc2_body,
    out_shape=jax.ShapeDtypeStruct((N_NODES, F), jnp.float32),
)

_tc3_call = pl.pallas_call(
    _tc3_body,
    out_shape=jax.ShapeDtypeStruct((N_NODES, F), jnp.float32),
)


def kernel(x, edge_index, W1, b1, W2, b2):
    sc1_call, sc2_call = _sc_calls()
    src = edge_index[0].reshape(WORKERS * CPT, CH)
    dst = edge_index[1].reshape(WORKERS * CPT, CH)
    zeros1 = jnp.zeros((NPAD,), jnp.float32)
    zeros16 = jnp.zeros((NPAD, F), jnp.float32)

    h1 = _tc_mm_call(x, W1)
    dis, agg1 = sc1_call(h1, dst, src, dst, zeros1, zeros16)
    diss = dis[:N_NODES, None]

    h2p = _tc2_call(agg1, h1, diss, b1.reshape(1, F), W2)
    agg2 = sc2_call(h2p, src, dst, zeros16)
    return _tc3_call(agg2, h2p, diss, b2.reshape(1, F))
